# Initial kernel scaffold; baseline (speedup 1.0000x reference)
import jax
import jax.numpy as jnp
from jax.experimental import pallas as pl


def kernel(x_edges, x_edges_values, x_nodes, x_nodes_coord, edge_embed, node_embed, W_dist, W_coord, W_msg, b_msg, W_node, b_node, W_cls, b_cls, logit_noedge):
    raise NotImplementedError("write your pallas kernel here")


# speedup vs baseline: 23.7397x; 23.7397x over previous
import jax
import jax.numpy as jnp
from jax.experimental import pallas as pl


def _gcn_fused_kernel(xe_ref, xev_ref, xnc_ref, ee_ref, ne_ref, wd_ref,
                      wco_ref, wm_ref, bm_ref, wn_ref, bn_ref, wc_ref,
                      bc_ref, out_ref):
    N = xe_ref.shape[1]
    H = ne_ref.shape[1]
    xe = xe_ref[0]
    xev = xev_ref[0]
    xnc = xnc_ref[0]

    h = xnc @ wco_ref[...] + ne_ref[0][None, :]

    xe3 = xe[:, :, None]
    e = jnp.where(xe3 == 0, ee_ref[0][None, None, :],
                  jnp.where(xe3 == 1, ee_ref[1][None, None, :],
                            ee_ref[2][None, None, :]))
    e = e + xev[:, :, None] * wd_ref[0][None, None, :]

    wm = wm_ref[...]
    a_src = h @ wm[H:2 * H]
    a_dst = h @ wm[2 * H:]
    pre = (e.reshape(N * N, H) @ wm[:H]).reshape(N, N, H)
    pre = pre + a_src[:, None, :] + a_dst[None, :, :] + bm_ref[0][None, None, :]
    m = jnp.maximum(pre, 0.0)
    gm = jax.nn.sigmoid(e) * m
    agg = jnp.sum(gm, axis=0)

    h_new = jnp.maximum(h + agg @ wn_ref[...] + bn_ref[0][None, :], 0.0)

    wc = wc_ref[...]
    t_src = h_new @ wc[H:2 * H]
    t_dst = h_new @ wc[2 * H:]
    e_new = jnp.maximum(e + m, 0.0)
    y = (e_new.reshape(N * N, H) @ wc[:H]).reshape(N, N, 2)
    y = y + t_src[:, None, :] + t_dst[None, :, :] + bc_ref[0][None, None, :]
    out_ref[0] = y


@jax.jit
def kernel(x_edges, x_edges_values, x_nodes, x_nodes_coord, edge_embed,
           node_embed, W_dist, W_coord, W_msg, b_msg, W_node, b_node,
           W_cls, b_cls, logit_noedge):
    B, N = x_nodes.shape
    H = node_embed.shape[1]
    C = W_cls.shape[1]
    full = lambda shape: pl.BlockSpec(shape, lambda b: (0,) * len(shape))
    out = pl.pallas_call(
        _gcn_fused_kernel,
        grid=(B,),
        in_specs=[
            pl.BlockSpec((1, N, N), lambda b: (b, 0, 0)),
            pl.BlockSpec((1, N, N), lambda b: (b, 0, 0)),
            pl.BlockSpec((1, N, 2), lambda b: (b, 0, 0)),
            full((3, H)),
            full((1, H)),
            full((1, H)),
            full((2, H)),
            full((3 * H, H)),
            full((1, H)),
            full((H, H)),
            full((1, H)),
            full((3 * H, C)),
            full((1, C)),
        ],
        out_specs=pl.BlockSpec((1, N, N, C), lambda b: (b, 0, 0, 0)),
        out_shape=jax.ShapeDtypeStruct((B, N, N, C), jnp.float32),
    )(x_edges, x_edges_values, x_nodes_coord, edge_embed, node_embed,
      W_dist, W_coord, W_msg, b_msg.reshape(1, H), W_node,
      b_node.reshape(1, H), W_cls, b_cls.reshape(1, C))
    return out
